# baseline (speedup 1.0000x reference)
import functools

import jax
import jax.numpy as jnp
from jax import lax
from jax.experimental import pallas as pl
from jax.experimental.pallas import tpu as pltpu
from jax.experimental.pallas import tpu_sc as plsc

_NC, _NS, _L = 2, 16, 16
_NW = _NC * _NS
_CH = 16


@functools.lru_cache(maxsize=None)
def _make_sc_kernel(B, S, D):
    rows_w = S // _NW
    nch = rows_w // _CH
    blk = _CH * D
    mesh = plsc.VectorSubcoreMesh(core_axis_name="c", subcore_axis_name="s")
    buf = pltpu.VMEM((blk,), jnp.float32)
    sem = pltpu.SemaphoreType.DMA

    @functools.partial(
        pl.kernel,
        out_type=jax.ShapeDtypeStruct((B, S * D), jnp.float32),
        mesh=mesh,
        scratch_types=[buf] * (2 + B) + [sem] * (2 + 2 * B),
    )
    def sc_add(x_hbm, pe_hbm, out_hbm, *scratch):
        pe_bufs = tuple(zip(scratch[:2], scratch[2 + B : 4 + B]))
        x_refs = scratch[2 : 2 + B]
        in_sems = scratch[4 + B : 4 + 2 * B]
        out_sems = scratch[4 + 2 * B : 4 + 3 * B]

        wid = lax.axis_index("s") * _NC + lax.axis_index("c")
        base = wid * rows_w * D

        pltpu.async_copy(pe_hbm.at[pl.ds(base, blk)], pe_bufs[0][0], pe_bufs[0][1])
        for b in range(B):
            pltpu.async_copy(x_hbm.at[b, pl.ds(base, blk)], x_refs[b], in_sems[b])

        def chunk_pair(ci2, carry):
            for cpar in (0, 1):
                ci = ci2 * 2 + cpar
                s0 = base + ci * blk
                peb, pes = pe_bufs[cpar]
                pltpu.make_async_copy(pe_hbm.at[pl.ds(s0, blk)], peb, pes).wait()

                @pl.when(ci + 1 < nch)
                def _():
                    nb, ns = pe_bufs[1 - cpar]
                    pltpu.async_copy(pe_hbm.at[pl.ds(s0 + blk, blk)], nb, ns)

                for b in range(B):
                    xb = x_refs[b]
                    pltpu.make_async_copy(
                        x_hbm.at[b, pl.ds(s0, blk)], xb, in_sems[b]
                    ).wait()

                    @plsc.parallel_loop(0, blk, step=_L, unroll=8)
                    def _vec(i):
                        plsc.addupdate(xb.at[pl.ds(i, _L)], peb[pl.ds(i, _L)])

                    pltpu.async_copy(xb, out_hbm.at[b, pl.ds(s0, blk)], out_sems[b])

                @pl.when(ci + 1 < nch)
                def _():
                    for b in range(B):
                        xb = x_refs[b]
                        pltpu.make_async_copy(
                            xb, out_hbm.at[b, pl.ds(s0, blk)], out_sems[b]
                        ).wait()
                        pltpu.async_copy(
                            x_hbm.at[b, pl.ds(s0 + blk, blk)], xb, in_sems[b]
                        )

            return carry

        lax.fori_loop(0, nch // 2, chunk_pair, 0)

        last = base + (nch - 1) * blk
        for b in range(B):
            pltpu.make_async_copy(
                x_refs[b], out_hbm.at[b, pl.ds(last, blk)], out_sems[b]
            ).wait()

    return sc_add


def kernel(x, pe_weight):
    B, S, D = x.shape
    out = _make_sc_kernel(B, S, D)(
        x.reshape(B, S * D), pe_weight[:S].reshape(S * D)
    )
    return out.reshape(B, S, D)


# speedup vs baseline: 4.0039x; 4.0039x over previous
import jax
import jax.numpy as jnp
from jax.experimental import pallas as pl


_BLK_S = 2048


def _add_kernel(x_ref, pe_ref, o_ref):
    o_ref[...] = x_ref[...] + pe_ref[...]


def kernel(x, pe_weight):
    batch, seq_len, d_model = x.shape
    pe = pe_weight[:seq_len]
    grid = (seq_len // _BLK_S, batch)
    return pl.pallas_call(
        _add_kernel,
        grid=grid,
        in_specs=[
            pl.BlockSpec((1, _BLK_S, d_model), lambda i, b: (b, i, 0)),
            pl.BlockSpec((_BLK_S, d_model), lambda i, b: (i, 0)),
        ],
        out_specs=pl.BlockSpec((1, _BLK_S, d_model), lambda i, b: (b, i, 0)),
        out_shape=jax.ShapeDtypeStruct(x.shape, x.dtype),
    )(x, pe)
